# baseline (speedup 1.0000x reference)
import jax
import jax.numpy as jnp
from jax import lax
from jax.experimental import pallas as pl
from jax.experimental.pallas import tpu as pltpu
from jax.experimental.pallas import tpu_sc as plsc

VOCAB = 100000
EMBED_DIM = 128
BATCH = 4096
HIST = 200

NUM_CORES = 2
NUM_SUBCORES = 16
NUM_WORKERS = NUM_CORES * NUM_SUBCORES

B = BATCH * HIST
B_PER_W = B // NUM_WORKERS
CHUNK = 128
N_CHUNKS = B_PER_W // CHUNK
NBUF = 5


def _gather_body(table_hbm, idx_hbm, out_hbm, idx_v, rows_v, gsem, ssem):
    wid = lax.axis_index("s") * NUM_CORES + lax.axis_index("c")
    base = wid * B_PER_W
    pltpu.sync_copy(idx_hbm.at[pl.ds(base, B_PER_W)], idx_v)

    def start_gather(j, b):
        pltpu.async_copy(
            table_hbm.at[idx_v.at[pl.ds(j * CHUNK, CHUNK)]],
            rows_v.at[b],
            gsem.at[b],
        )

    def start_store(j, b):
        pltpu.async_copy(
            rows_v.at[b], out_hbm.at[pl.ds(base + j * CHUNK, CHUNK)], ssem.at[b]
        )

    def wait_gather(b):
        pltpu.make_async_copy(
            out_hbm.at[pl.ds(base, CHUNK)], rows_v.at[b], gsem.at[b]
        ).wait()

    def wait_store(b):
        pltpu.make_async_copy(
            rows_v.at[b], out_hbm.at[pl.ds(base, CHUNK)], ssem.at[b]
        ).wait()

    for b in range(NBUF):
        start_gather(b, b)

    @pl.loop(0, N_CHUNKS - NBUF, step=NBUF)
    def _group(j0):
        for b in range(NBUF):
            wait_gather(b)
            start_store(j0 + b, b)
        for b in range(NBUF):
            wait_store(b)
            start_gather(j0 + b + NBUF, b)

    for b in range(NBUF):
        wait_gather(b)
        start_store(N_CHUNKS - NBUF + b, b)
    for b in range(NBUF):
        wait_store(b)


@jax.jit
def _embedding_lookup(idx_flat, table):
    mesh = plsc.VectorSubcoreMesh(
        core_axis_name="c",
        subcore_axis_name="s",
        num_cores=NUM_CORES,
        num_subcores=NUM_SUBCORES,
    )
    run = pl.kernel(
        _gather_body,
        out_type=jax.ShapeDtypeStruct((B, EMBED_DIM), jnp.float32),
        mesh=mesh,
        scratch_types=[
            pltpu.VMEM((B_PER_W,), jnp.int32),
            pltpu.VMEM((NBUF, CHUNK, EMBED_DIM), jnp.float32),
            pltpu.SemaphoreType.DMA((NBUF,)),
            pltpu.SemaphoreType.DMA((NBUF,)),
        ],
    )
    return run(table, idx_flat)


def kernel(idx, table):
    idx_flat = idx.reshape(B).astype(jnp.int32)
    out = _embedding_lookup(idx_flat, table)
    return out.reshape(BATCH, HIST, EMBED_DIM)


# speedup vs baseline: 1.0071x; 1.0071x over previous
import jax
import jax.numpy as jnp
from jax import lax
from jax.experimental import pallas as pl
from jax.experimental.pallas import tpu as pltpu
from jax.experimental.pallas import tpu_sc as plsc

VOCAB = 100000
EMBED_DIM = 128
BATCH = 4096
HIST = 200

NUM_CORES = 2
NUM_SUBCORES = 16
NUM_WORKERS = NUM_CORES * NUM_SUBCORES

B = BATCH * HIST
B_PER_W = B // NUM_WORKERS
CHUNK = 64
N_CHUNKS = B_PER_W // CHUNK
NBUF = 8


def _gather_body(table_hbm, idx_hbm, out_hbm, idx_v, rows_v, gsem, ssem):
    wid = lax.axis_index("s") * NUM_CORES + lax.axis_index("c")
    base = wid * B_PER_W
    pltpu.sync_copy(idx_hbm.at[pl.ds(base, B_PER_W)], idx_v)

    def start_gather(j, b):
        pltpu.async_copy(
            table_hbm.at[idx_v.at[pl.ds(j * CHUNK, CHUNK)]],
            rows_v.at[b],
            gsem.at[b],
        )

    def start_store(j, b):
        pltpu.async_copy(
            rows_v.at[b], out_hbm.at[pl.ds(base + j * CHUNK, CHUNK)], ssem.at[b]
        )

    def wait_gather(b):
        pltpu.make_async_copy(
            out_hbm.at[pl.ds(base, CHUNK)], rows_v.at[b], gsem.at[b]
        ).wait()

    def wait_store(b):
        pltpu.make_async_copy(
            rows_v.at[b], out_hbm.at[pl.ds(base, CHUNK)], ssem.at[b]
        ).wait()

    for b in range(NBUF):
        start_gather(b, b)

    @pl.loop(0, N_CHUNKS - NBUF, step=NBUF)
    def _group(j0):
        for b in range(NBUF):
            wait_gather(b)
            start_store(j0 + b, b)
        for b in range(NBUF):
            wait_store(b)
            start_gather(j0 + b + NBUF, b)

    for b in range(NBUF):
        wait_gather(b)
        start_store(N_CHUNKS - NBUF + b, b)
    for b in range(NBUF):
        wait_store(b)


@jax.jit
def _embedding_lookup(idx_flat, table):
    mesh = plsc.VectorSubcoreMesh(
        core_axis_name="c",
        subcore_axis_name="s",
        num_cores=NUM_CORES,
        num_subcores=NUM_SUBCORES,
    )
    run = pl.kernel(
        _gather_body,
        out_type=jax.ShapeDtypeStruct((B, EMBED_DIM), jnp.float32),
        mesh=mesh,
        scratch_types=[
            pltpu.VMEM((B_PER_W,), jnp.int32),
            pltpu.VMEM((NBUF, CHUNK, EMBED_DIM), jnp.float32),
            pltpu.SemaphoreType.DMA((NBUF,)),
            pltpu.SemaphoreType.DMA((NBUF,)),
        ],
    )
    return run(table, idx_flat)


def kernel(idx, table):
    idx_flat = idx.reshape(B).astype(jnp.int32)
    out = _embedding_lookup(idx_flat, table)
    return out.reshape(BATCH, HIST, EMBED_DIM)
